# baseline (speedup 1.0000x reference)
import jax
import jax.numpy as jnp
from jax.experimental import pallas as pl
from jax.experimental.pallas import tpu as pltpu

_ROWS = 2048
_WIDTH = 1024
_BLK = 256


def _body(sym_ref, in_ref, tab_ref, last_ref, out_ref, emd_ref):
    b = pl.program_id(1)

    @pl.when(b == 0)
    def _compute_emd():
        r = pl.program_id(0)
        sym = sym_ref[0]
        rows = r * _BLK + jax.lax.broadcasted_iota(jnp.int32, (_BLK, 1), 0)
        emd = jnp.where(rows < sym, tab_ref[...], last_ref[...])
        emd_ref[...] = jnp.where(rows == sym, jnp.float32(0.0), emd)

    out_ref[...] = in_ref[...] + emd_ref[...][None]


def kernel(inputs, sym_index, table):
    batch = inputs.shape[0]
    sym = jnp.asarray(sym_index, jnp.int32).reshape(1)
    last = table[-1:, :]
    grid = (_ROWS // _BLK, batch)
    return pl.pallas_call(
        _body,
        grid_spec=pltpu.PrefetchScalarGridSpec(
            num_scalar_prefetch=1,
            grid=grid,
            in_specs=[
                pl.BlockSpec((1, _BLK, _WIDTH), lambda r, b, sym: (b, r, 0)),
                pl.BlockSpec((_BLK, _WIDTH), lambda r, b, sym: (r, 0)),
                pl.BlockSpec((1, _WIDTH), lambda r, b, sym: (0, 0)),
            ],
            out_specs=pl.BlockSpec((1, _BLK, _WIDTH), lambda r, b, sym: (b, r, 0)),
            scratch_shapes=[pltpu.VMEM((_BLK, _WIDTH), jnp.float32)],
        ),
        out_shape=jax.ShapeDtypeStruct(inputs.shape, inputs.dtype),
    )(sym, inputs, table, last)


# speedup vs baseline: 1.2887x; 1.2887x over previous
import jax
import jax.numpy as jnp
from jax.experimental import pallas as pl
from jax.experimental.pallas import tpu as pltpu

_ROWS = 2048
_WIDTH = 1024
_BLK = 512


def _body(sym_ref, in_ref, tab_ref, last_ref, out_ref, emd_ref):
    b = pl.program_id(1)

    @pl.when(b == 0)
    def _compute_emd():
        r = pl.program_id(0)
        sym = sym_ref[0]
        rows = r * _BLK + jax.lax.broadcasted_iota(jnp.int32, (_BLK, 1), 0)
        emd = jnp.where(rows < sym, tab_ref[...], last_ref[...])
        emd_ref[...] = jnp.where(rows == sym, jnp.float32(0.0), emd)

    out_ref[...] = in_ref[...] + emd_ref[...][None]


def kernel(inputs, sym_index, table):
    batch = inputs.shape[0]
    sym = jnp.asarray(sym_index, jnp.int32).reshape(1)
    last = table[-1:, :]
    grid = (_ROWS // _BLK, batch)
    return pl.pallas_call(
        _body,
        grid_spec=pltpu.PrefetchScalarGridSpec(
            num_scalar_prefetch=1,
            grid=grid,
            in_specs=[
                pl.BlockSpec((1, _BLK, _WIDTH), lambda r, b, sym: (b, r, 0)),
                pl.BlockSpec((_BLK, _WIDTH), lambda r, b, sym: (r, 0)),
                pl.BlockSpec((1, _WIDTH), lambda r, b, sym: (0, 0)),
            ],
            out_specs=pl.BlockSpec((1, _BLK, _WIDTH), lambda r, b, sym: (b, r, 0)),
            scratch_shapes=[pltpu.VMEM((_BLK, _WIDTH), jnp.float32)],
        ),
        out_shape=jax.ShapeDtypeStruct(inputs.shape, inputs.dtype),
    )(sym, inputs, table, last)


# speedup vs baseline: 1.3894x; 1.0781x over previous
import jax
import jax.numpy as jnp
from jax.experimental import pallas as pl
from jax.experimental.pallas import tpu as pltpu

_ROWS = 2048
_WIDTH = 1024
_BLK = 1024


def _body(sym_ref, in_ref, tab_ref, last_ref, out_ref, emd_ref):
    b = pl.program_id(1)

    @pl.when(b == 0)
    def _compute_emd():
        r = pl.program_id(0)
        sym = sym_ref[0]
        rows = r * _BLK + jax.lax.broadcasted_iota(jnp.int32, (_BLK, 1), 0)
        emd = jnp.where(rows < sym, tab_ref[...], last_ref[...])
        emd_ref[...] = jnp.where(rows == sym, jnp.float32(0.0), emd)

    out_ref[...] = in_ref[...] + emd_ref[...][None]


def kernel(inputs, sym_index, table):
    batch = inputs.shape[0]
    sym = jnp.asarray(sym_index, jnp.int32).reshape(1)
    last = table[-1:, :]
    grid = (_ROWS // _BLK, batch)
    return pl.pallas_call(
        _body,
        grid_spec=pltpu.PrefetchScalarGridSpec(
            num_scalar_prefetch=1,
            grid=grid,
            in_specs=[
                pl.BlockSpec((1, _BLK, _WIDTH), lambda r, b, sym: (b, r, 0)),
                pl.BlockSpec((_BLK, _WIDTH), lambda r, b, sym: (r, 0)),
                pl.BlockSpec((1, _WIDTH), lambda r, b, sym: (0, 0)),
            ],
            out_specs=pl.BlockSpec((1, _BLK, _WIDTH), lambda r, b, sym: (b, r, 0)),
            scratch_shapes=[pltpu.VMEM((_BLK, _WIDTH), jnp.float32)],
        ),
        out_shape=jax.ShapeDtypeStruct(inputs.shape, inputs.dtype),
    )(sym, inputs, table, last)


# speedup vs baseline: 1.4954x; 1.0763x over previous
import jax
import jax.numpy as jnp
from jax.experimental import pallas as pl
from jax.experimental.pallas import tpu as pltpu

_ROWS = 2048
_WIDTH = 1024
_BLK = 2048


def _body(sym_ref, in_ref, tab_ref, last_ref, out_ref, emd_ref):
    b = pl.program_id(1)

    @pl.when(b == 0)
    def _compute_emd():
        r = pl.program_id(0)
        sym = sym_ref[0]
        rows = r * _BLK + jax.lax.broadcasted_iota(jnp.int32, (_BLK, 1), 0)
        emd = jnp.where(rows < sym, tab_ref[...], last_ref[...])
        emd_ref[...] = jnp.where(rows == sym, jnp.float32(0.0), emd)

    out_ref[...] = in_ref[...] + emd_ref[...][None]


def kernel(inputs, sym_index, table):
    batch = inputs.shape[0]
    sym = jnp.asarray(sym_index, jnp.int32).reshape(1)
    last = table[-1:, :]
    grid = (_ROWS // _BLK, batch)
    return pl.pallas_call(
        _body,
        grid_spec=pltpu.PrefetchScalarGridSpec(
            num_scalar_prefetch=1,
            grid=grid,
            in_specs=[
                pl.BlockSpec((1, _BLK, _WIDTH), lambda r, b, sym: (b, r, 0)),
                pl.BlockSpec((_BLK, _WIDTH), lambda r, b, sym: (r, 0)),
                pl.BlockSpec((1, _WIDTH), lambda r, b, sym: (0, 0)),
            ],
            out_specs=pl.BlockSpec((1, _BLK, _WIDTH), lambda r, b, sym: (b, r, 0)),
            scratch_shapes=[pltpu.VMEM((_BLK, _WIDTH), jnp.float32)],
        ),
        out_shape=jax.ShapeDtypeStruct(inputs.shape, inputs.dtype),
    )(sym, inputs, table, last)
